# baseline (speedup 1.0000x reference)
import functools

import numpy as np

import jax
import jax.numpy as jnp
from jax import lax
from jax.experimental import pallas as pl
from jax.experimental.pallas import tpu as pltpu
from jax.experimental.pallas import tpu_sc as plsc

N_NODES = 10000
N_EDGES = 160000
D_IN = 256
D_OUT = 256
DH = 128
NS = 16
CH = 80
NCH = 2048
E_PAD = NCH * CH
BCH = 32
NBLK = NCH // (NS * BCH)
RPT = N_NODES // NS
WRB = 624
WRB_TAIL = N_NODES - (NS - 1) * WRB
MM_BLK = 2000

_PERM = np.empty((D_OUT,), np.int32)
for _c in range(2):
    for _q in range(DH // 32):
        for _i in range(16):
            _PERM[_c * DH + 32 * _q + 2 * _i] = _c * DH + 32 * _q + _i
            _PERM[_c * DH + 32 * _q + 2 * _i + 1] = _c * DH + 32 * _q + 16 + _i


def _mm_body(x_ref, w_ref, b_ref, o_ref):
    o_ref[0, :, :] = (
        jnp.dot(x_ref[...], w_ref[...], preferred_element_type=jnp.float32)
        + b_ref[...]
    ).astype(jnp.bfloat16)


def _matmul_halves(x, W, bias):
    grid = (2, N_NODES // MM_BLK)
    return pl.pallas_call(
        _mm_body,
        grid=grid,
        in_specs=[
            pl.BlockSpec((MM_BLK, D_IN), lambda c, i: (i, 0)),
            pl.BlockSpec((D_IN, DH), lambda c, i: (0, c)),
            pl.BlockSpec((1, DH), lambda c, i: (0, c)),
        ],
        out_specs=pl.BlockSpec((1, MM_BLK, DH), lambda c, i: (c, i, 0)),
        out_shape=jax.ShapeDtypeStruct((2, N_NODES, DH), jnp.bfloat16),
    )(x, W, bias.reshape(1, D_OUT))


_mesh = plsc.VectorSubcoreMesh(core_axis_name="c", subcore_axis_name="s")


@functools.partial(
    pl.kernel,
    out_type=jax.ShapeDtypeStruct((N_NODES, D_OUT), jnp.float32),
    mesh=_mesh,
    compiler_params=pltpu.CompilerParams(needs_layout_passes=False, use_tc_tiling_on_sc=False),
    scratch_types=[
        pltpu.VMEM((BCH, CH), jnp.int32),
        pltpu.VMEM((BCH, CH), jnp.int32),
        pltpu.VMEM((BCH, CH), jnp.float32),
        pltpu.VMEM((CH, DH // 2), jnp.int32),
        pltpu.VMEM((CH, DH // 2), jnp.int32),
        pltpu.VMEM((CH, DH), jnp.float32),
        pltpu.VMEM((CH, DH), jnp.float32),
        pltpu.VMEM_SHARED((N_NODES, DH), jnp.float32),
        pltpu.SemaphoreType.DMA,
        pltpu.SemaphoreType.DMA,
        pltpu.SemaphoreType.DMA,
        pltpu.SemaphoreType.DMA,
    ],
)
def _aggregate(h_hbm, src_hbm, dst_hbm, w_hbm, out_hbm,
               src2d, dst2d, w2d, buf_a, buf_b, rf_a, rf_b, acc_sh,
               sem_a, sem_b, ssem_a, ssem_b):
    c = lax.axis_index("c")
    s = lax.axis_index("s")

    z16 = jnp.zeros((16,), jnp.float32)

    def zrow(i, carry):
        for j in range(DH // 16):
            rf_a[i, pl.ds(16 * j, 16)] = z16
        return carry

    lax.fori_loop(0, CH, zrow, 0)
    for t in range(RPT // CH):
        pltpu.sync_copy(rf_a, acc_sh.at[pl.ds(s * RPT + t * CH, CH)])
    pltpu.sync_copy(rf_a.at[pl.ds(0, RPT % CH)],
                    acc_sh.at[pl.ds(s * RPT + (RPT // CH) * CH, RPT % CH)])
    plsc.subcore_barrier()

    def gather_start(jj, buf, sem):
        return pltpu.async_copy(h_hbm.at[src2d.at[jj]], buf, sem)

    def gather_wait(buf, sem):
        pltpu.make_async_copy(h_hbm.at[src2d.at[0]], buf, sem).wait()

    def scale(jj, buf, rf):
        for l in range(CH // 16):
            w16 = w2d[jj, pl.ds(l * 16, 16)]
            for i in range(16):
                e = l * 16 + i
                wspl = jnp.full((16,), w16[i], jnp.float32)
                for q in range(DH // 32):
                    v16 = buf[e, pl.ds(16 * q, 16)]
                    v = plsc.bitcast(v16, jnp.bfloat16)
                    a, b = plsc.unpack(v, format=plsc.PackFormat.INTERLEAVED)
                    rf[e, pl.ds(32 * q, 16)] = a * wspl
                    rf[e, pl.ds(32 * q + 16, 16)] = b * wspl

    def scatter_start(jj, rf, ssem):
        pltpu.async_copy(rf, acc_sh.at[dst2d.at[jj]], ssem, add=True)

    def scatter_wait(rf, ssem):
        pltpu.make_async_copy(rf, acc_sh.at[dst2d.at[0]], ssem).wait()

    def block(o, carry):
        row_base = s * (NBLK * BCH) + o * BCH
        pltpu.sync_copy(src_hbm.at[c, pl.ds(row_base, BCH)], src2d)
        pltpu.sync_copy(dst_hbm.at[pl.ds(row_base, BCH)], dst2d)
        pltpu.sync_copy(w_hbm.at[pl.ds(row_base, BCH)], w2d)
        gather_start(0, buf_a, sem_a)
        gather_start(1, buf_b, sem_b)
        gather_wait(buf_a, sem_a)
        scale(0, buf_a, rf_a)
        scatter_start(0, rf_a, ssem_a)
        gather_start(2, buf_a, sem_a)
        gather_wait(buf_b, sem_b)
        scale(1, buf_b, rf_b)
        scatter_start(1, rf_b, ssem_b)
        gather_start(3, buf_b, sem_b)

        def pair(p, pcarry):
            a = 2 * p
            gather_wait(buf_a, sem_a)
            scatter_wait(rf_a, ssem_a)
            scale(a, buf_a, rf_a)
            scatter_start(a, rf_a, ssem_a)

            @pl.when(a + 2 < BCH)
            def _():
                gather_start(a + 2, buf_a, sem_a)

            gather_wait(buf_b, sem_b)
            scatter_wait(rf_b, ssem_b)
            scale(a + 1, buf_b, rf_b)
            scatter_start(a + 1, rf_b, ssem_b)

            @pl.when(a + 3 < BCH)
            def _():
                gather_start(a + 3, buf_b, sem_b)

            return pcarry

        lax.fori_loop(1, BCH // 2, pair, 0)
        scatter_wait(rf_a, ssem_a)
        scatter_wait(rf_b, ssem_b)
        return carry

    lax.fori_loop(0, NBLK, block, 0)
    plsc.subcore_barrier()

    pltpu.sync_copy(acc_sh.at[pl.ds(s * RPT, RPT)],
                    out_hbm.at[pl.ds(s * RPT, RPT), pl.ds(c * DH, DH)])


def kernel(x, edge_index, edge_weight, W, bias):
    ei = edge_index.astype(jnp.int32)
    npad = E_PAD - N_EDGES
    zpad = jnp.zeros((npad,), jnp.int32)
    dst = jnp.concatenate([ei[0], zpad]).reshape(NCH, CH)
    src = jnp.concatenate([ei[1], zpad])
    src01 = jnp.stack([src, src + N_NODES]).reshape(2, NCH, CH)
    w3 = jnp.concatenate(
        [edge_weight, jnp.zeros((npad,), jnp.float32)]).reshape(NCH, CH)
    perm = jnp.asarray(_PERM)
    h2 = _matmul_halves(x, W[:, perm], bias[perm])
    h_flat = jax.lax.bitcast_convert_type(
        h2.reshape(2 * N_NODES, DH // 2, 2), jnp.int32)
    return _aggregate(h_flat, src01, dst, w3)


# speedup vs baseline: 1.0216x; 1.0216x over previous
import functools

import numpy as np

import jax
import jax.numpy as jnp
from jax import lax
from jax.experimental import pallas as pl
from jax.experimental.pallas import tpu as pltpu
from jax.experimental.pallas import tpu_sc as plsc

N_NODES = 10000
N_EDGES = 160000
D_IN = 256
D_OUT = 256
DH = 128
NS = 16
CH = 80
NCH = 2048
E_PAD = NCH * CH
BCH = 64
NBLK = NCH // (NS * BCH)
RPT = N_NODES // NS
WRB = 624
WRB_TAIL = N_NODES - (NS - 1) * WRB
MM_BLK = 2000

_PERM = np.empty((D_OUT,), np.int32)
for _c in range(2):
    for _q in range(DH // 32):
        for _i in range(16):
            _PERM[_c * DH + 32 * _q + 2 * _i] = _c * DH + 32 * _q + _i
            _PERM[_c * DH + 32 * _q + 2 * _i + 1] = _c * DH + 32 * _q + 16 + _i


def _mm_body(x_ref, w_ref, b_ref, o_ref):
    o_ref[0, :, :] = (
        jnp.dot(x_ref[...], w_ref[...], preferred_element_type=jnp.float32)
        + b_ref[...]
    ).astype(jnp.bfloat16)


def _matmul_halves(x, W, bias):
    grid = (2, N_NODES // MM_BLK)
    return pl.pallas_call(
        _mm_body,
        grid=grid,
        in_specs=[
            pl.BlockSpec((MM_BLK, D_IN), lambda c, i: (i, 0)),
            pl.BlockSpec((D_IN, DH), lambda c, i: (0, c)),
            pl.BlockSpec((1, DH), lambda c, i: (0, c)),
        ],
        out_specs=pl.BlockSpec((1, MM_BLK, DH), lambda c, i: (c, i, 0)),
        out_shape=jax.ShapeDtypeStruct((2, N_NODES, DH), jnp.bfloat16),
    )(x, W, bias.reshape(1, D_OUT))


_mesh = plsc.VectorSubcoreMesh(core_axis_name="c", subcore_axis_name="s")


@functools.partial(
    pl.kernel,
    out_type=jax.ShapeDtypeStruct((N_NODES, D_OUT), jnp.float32),
    mesh=_mesh,
    compiler_params=pltpu.CompilerParams(needs_layout_passes=False, use_tc_tiling_on_sc=False),
    scratch_types=[
        pltpu.VMEM((BCH, CH), jnp.int32),
        pltpu.VMEM((BCH, CH), jnp.int32),
        pltpu.VMEM((BCH, CH), jnp.float32),
        pltpu.VMEM((CH, DH // 2), jnp.int32),
        pltpu.VMEM((CH, DH // 2), jnp.int32),
        pltpu.VMEM((CH, DH), jnp.float32),
        pltpu.VMEM((CH, DH), jnp.float32),
        pltpu.VMEM_SHARED((N_NODES, DH), jnp.float32),
        pltpu.SemaphoreType.DMA,
        pltpu.SemaphoreType.DMA,
        pltpu.SemaphoreType.DMA,
        pltpu.SemaphoreType.DMA,
    ],
)
def _aggregate(h_hbm, src_hbm, dst_hbm, w_hbm, out_hbm,
               src2d, dst2d, w2d, buf_a, buf_b, rf_a, rf_b, acc_sh,
               sem_a, sem_b, ssem_a, ssem_b):
    c = lax.axis_index("c")
    s = lax.axis_index("s")

    z16 = jnp.zeros((16,), jnp.float32)

    def zrow(i, carry):
        for j in range(DH // 16):
            rf_a[i, pl.ds(16 * j, 16)] = z16
        return carry

    lax.fori_loop(0, CH, zrow, 0)
    for t in range(RPT // CH):
        pltpu.sync_copy(rf_a, acc_sh.at[pl.ds(s * RPT + t * CH, CH)])
    pltpu.sync_copy(rf_a.at[pl.ds(0, RPT % CH)],
                    acc_sh.at[pl.ds(s * RPT + (RPT // CH) * CH, RPT % CH)])
    plsc.subcore_barrier()

    def gather_start(jj, buf, sem):
        return pltpu.async_copy(h_hbm.at[src2d.at[jj]], buf, sem)

    def gather_wait(buf, sem):
        pltpu.make_async_copy(h_hbm.at[src2d.at[0]], buf, sem).wait()

    def scale(jj, buf, rf):
        for l in range(CH // 16):
            w16 = w2d[jj, pl.ds(l * 16, 16)]
            for i in range(16):
                e = l * 16 + i
                wspl = jnp.full((16,), w16[i], jnp.float32)
                for q in range(DH // 32):
                    v16 = buf[e, pl.ds(16 * q, 16)]
                    v = plsc.bitcast(v16, jnp.bfloat16)
                    a, b = plsc.unpack(v, format=plsc.PackFormat.INTERLEAVED)
                    rf[e, pl.ds(32 * q, 16)] = a * wspl
                    rf[e, pl.ds(32 * q + 16, 16)] = b * wspl

    def scatter_start(jj, rf, ssem):
        pltpu.async_copy(rf, acc_sh.at[dst2d.at[jj]], ssem, add=True)

    def scatter_wait(rf, ssem):
        pltpu.make_async_copy(rf, acc_sh.at[dst2d.at[0]], ssem).wait()

    def block(o, carry):
        row_base = s * (NBLK * BCH) + o * BCH
        pltpu.sync_copy(src_hbm.at[c, pl.ds(row_base, BCH)], src2d)
        pltpu.sync_copy(dst_hbm.at[pl.ds(row_base, BCH)], dst2d)
        pltpu.sync_copy(w_hbm.at[pl.ds(row_base, BCH)], w2d)
        gather_start(0, buf_a, sem_a)
        gather_start(1, buf_b, sem_b)
        gather_wait(buf_a, sem_a)
        scale(0, buf_a, rf_a)
        scatter_start(0, rf_a, ssem_a)
        gather_start(2, buf_a, sem_a)
        gather_wait(buf_b, sem_b)
        scale(1, buf_b, rf_b)
        scatter_start(1, rf_b, ssem_b)
        gather_start(3, buf_b, sem_b)

        def pair(p, pcarry):
            a = 2 * p
            gather_wait(buf_a, sem_a)
            scatter_wait(rf_a, ssem_a)
            scale(a, buf_a, rf_a)
            scatter_start(a, rf_a, ssem_a)

            @pl.when(a + 2 < BCH)
            def _():
                gather_start(a + 2, buf_a, sem_a)

            gather_wait(buf_b, sem_b)
            scatter_wait(rf_b, ssem_b)
            scale(a + 1, buf_b, rf_b)
            scatter_start(a + 1, rf_b, ssem_b)

            @pl.when(a + 3 < BCH)
            def _():
                gather_start(a + 3, buf_b, sem_b)

            return pcarry

        lax.fori_loop(1, BCH // 2, pair, 0)
        scatter_wait(rf_a, ssem_a)
        scatter_wait(rf_b, ssem_b)
        return carry

    lax.fori_loop(0, NBLK, block, 0)
    plsc.subcore_barrier()

    pltpu.sync_copy(acc_sh.at[pl.ds(s * RPT, RPT)],
                    out_hbm.at[pl.ds(s * RPT, RPT), pl.ds(c * DH, DH)])


def kernel(x, edge_index, edge_weight, W, bias):
    ei = edge_index.astype(jnp.int32)
    npad = E_PAD - N_EDGES
    zpad = jnp.zeros((npad,), jnp.int32)
    dst = jnp.concatenate([ei[0], zpad]).reshape(NCH, CH)
    src = jnp.concatenate([ei[1], zpad])
    src01 = jnp.stack([src, src + N_NODES]).reshape(2, NCH, CH)
    w3 = jnp.concatenate(
        [edge_weight, jnp.zeros((npad,), jnp.float32)]).reshape(NCH, CH)
    perm = jnp.asarray(_PERM)
    h2 = _matmul_halves(x, W[:, perm], bias[perm])
    h_flat = jax.lax.bitcast_convert_type(
        h2.reshape(2 * N_NODES, DH // 2, 2), jnp.int32)
    return _aggregate(h_flat, src01, dst, w3)
